# baseline (device time: 76552 ns/iter reference)
import jax
import jax.numpy as jnp
from jax import lax
from jax.experimental import pallas as pl
from jax.experimental.pallas import tpu as pltpu

NEG = -1e30
N_CHUNK = 4
NQ = 4


def kernel(x, router, W1, W2):
    T, D = x.shape
    E_loc = router.shape[1]
    F = W1.shape[2]
    H = T // 2
    C = H // N_CHUNK
    Q1 = D // NQ
    Q2 = F // NQ
    NQTOT = E_loc * NQ

    def body(x_ref, r_ref, w1_hbm, w2_hbm, out_ref,
             r_recv, xcat, xcat_bf, p_send, p_recv,
             w1_bf, w2_bf, stg1, stg2,
             sem_recv, sem_send, sem_pc, sem_ps, sem_oc, sem_os,
             sem_s1, sem_s2):
        my_x = lax.axis_index("x")
        my_y = lax.axis_index("y")
        xnbr = (1 - my_x, my_y)
        ynbr = (my_x, 1 - my_y)

        def q_cps(q):
            e, i = divmod(q, NQ)
            slot = q % 2
            return (
                pltpu.make_async_copy(w1_hbm.at[e, pl.ds(i * Q1, Q1)],
                                      stg1.at[slot], sem_s1.at[slot]),
                pltpu.make_async_copy(w2_hbm.at[e, pl.ds(i * Q2, Q2)],
                                      stg2.at[slot], sem_s2.at[slot]),
            )

        def start_q(q):
            for cp in q_cps(q):
                cp.start()

        start_q(0)
        start_q(1)

        xcat[pl.ds(0, H), :] = x_ref[pl.ds(my_y * H, H), :]

        barrier = pltpu.get_barrier_semaphore()
        for nbr in (xnbr, ynbr):
            pl.semaphore_signal(barrier, inc=1, device_id=nbr,
                                device_id_type=pl.DeviceIdType.MESH)
        pl.semaphore_wait(barrier, 2)

        rdma_r = pltpu.make_async_remote_copy(
            src_ref=r_ref, dst_ref=r_recv,
            send_sem=sem_send.at[0], recv_sem=sem_recv.at[0],
            device_id=xnbr, device_id_type=pl.DeviceIdType.MESH)
        rdma_r.start()
        rdma_x = pltpu.make_async_remote_copy(
            src_ref=xcat.at[pl.ds(0, H)], dst_ref=xcat.at[pl.ds(H, H)],
            send_sem=sem_send.at[1], recv_sem=sem_recv.at[1],
            device_id=xnbr, device_id_type=pl.DeviceIdType.MESH)
        rdma_x.start()
        rdma_r.wait()

        def gate_weights(xb):
            g_loc = jnp.dot(xb, r_ref[...], preferred_element_type=jnp.float32,
                            precision=lax.Precision.HIGHEST)
            g_rem = jnp.dot(xb, r_recv[...], preferred_element_type=jnp.float32,
                            precision=lax.Precision.HIGHEST)
            m1 = jnp.maximum(jnp.max(g_loc, 1, keepdims=True),
                             jnp.max(g_rem, 1, keepdims=True))
            is1 = g_loc == m1
            g2_loc = jnp.where(is1, NEG, g_loc)
            g2_rem = jnp.where(g_rem == m1, NEG, g_rem)
            m2 = jnp.maximum(jnp.max(g2_loc, 1, keepdims=True),
                             jnp.max(g2_rem, 1, keepdims=True))
            is2 = g2_loc == m2
            b = jnp.exp(m2 - m1)
            return (jnp.where(is1, 1.0 / (1.0 + b), 0.0)
                    + jnp.where(is2, b / (1.0 + b), 0.0))

        w_lo = gate_weights(xcat[:H, :])
        xcat_bf[pl.ds(0, H), :] = xcat[:H, :].astype(jnp.bfloat16)

        def ffn(xb, e):
            h = jnp.maximum(
                jnp.dot(xb, w1_bf[e], preferred_element_type=jnp.float32),
                0.0)
            return jnp.dot(h.astype(jnp.bfloat16), w2_bf[e],
                           preferred_element_type=jnp.float32)

        acc_lo = None
        acc_hi = None
        w_hi = None
        for q in range(NQTOT):
            e, i = divmod(q, NQ)
            slot = q % 2
            cp1, cp2 = q_cps(q)
            cp1.wait()
            cp2.wait()
            w1_bf[e, pl.ds(i * Q1, Q1), :] = stg1[slot].astype(jnp.bfloat16)
            w2_bf[e, pl.ds(i * Q2, Q2), :] = stg2[slot].astype(jnp.bfloat16)
            if q + 2 < NQTOT:
                start_q(q + 2)
            if i < NQ - 1:
                continue
            if e == 0:
                acc_lo = ffn(xcat_bf[:H, :], 0) * w_lo[:, 0:1]
            elif e == 1:
                rdma_x.wait()
                w_hi = gate_weights(xcat[H:, :])
                xcat_bf[pl.ds(H, H), :] = xcat[H:, :].astype(jnp.bfloat16)
                acc_hi = ffn(xcat_bf[H:, :], 0) * w_hi[:, 0:1]
                acc_hi = acc_hi + ffn(xcat_bf[H:, :], 1) * w_hi[:, 1:2]
            else:
                acc_hi = acc_hi + ffn(xcat_bf[H:, :], e) * w_hi[:, e:e + 1]

        p_send[...] = acc_hi
        for c in range(N_CHUNK):
            pltpu.make_async_remote_copy(
                src_ref=p_send.at[pl.ds(c * C, C)],
                dst_ref=p_recv.at[pl.ds(c * C, C)],
                send_sem=sem_ps.at[c], recv_sem=sem_pc.at[c],
                device_id=xnbr, device_id_type=pl.DeviceIdType.MESH).start()

        for e in range(1, E_loc):
            acc_lo = acc_lo + ffn(xcat_bf[:H, :], e) * w_lo[:, e:e + 1]

        fwd = []
        for c in range(N_CHUNK):
            rdma_p = pltpu.make_async_remote_copy(
                src_ref=p_send.at[pl.ds(c * C, C)],
                dst_ref=p_recv.at[pl.ds(c * C, C)],
                send_sem=sem_ps.at[c], recv_sem=sem_pc.at[c],
                device_id=xnbr, device_id_type=pl.DeviceIdType.MESH)
            rdma_p.wait_recv()
            rows = pl.ds(my_y * H + c * C, C)
            out_ref[rows, :] = (acc_lo[c * C:(c + 1) * C, :]
                                + p_recv[c * C:(c + 1) * C, :])
            rdma_o = pltpu.make_async_remote_copy(
                src_ref=out_ref.at[rows], dst_ref=out_ref.at[rows],
                send_sem=sem_os.at[c], recv_sem=sem_oc.at[c],
                device_id=ynbr, device_id_type=pl.DeviceIdType.MESH)
            rdma_o.start()
            fwd.append(rdma_o)

        for c in range(N_CHUNK):
            rdma_p = pltpu.make_async_remote_copy(
                src_ref=p_send.at[pl.ds(c * C, C)],
                dst_ref=p_recv.at[pl.ds(c * C, C)],
                send_sem=sem_ps.at[c], recv_sem=sem_pc.at[c],
                device_id=xnbr, device_id_type=pl.DeviceIdType.MESH)
            rdma_p.wait_send()
            fwd[c].wait()

    return pl.pallas_call(
        body,
        out_shape=jax.ShapeDtypeStruct((T, D), jnp.float32),
        in_specs=[
            pl.BlockSpec(memory_space=pltpu.VMEM),
            pl.BlockSpec(memory_space=pltpu.VMEM),
            pl.BlockSpec(memory_space=pl.ANY),
            pl.BlockSpec(memory_space=pl.ANY),
        ],
        out_specs=pl.BlockSpec(memory_space=pltpu.VMEM),
        scratch_shapes=[
            pltpu.VMEM((D, E_loc), jnp.float32),
            pltpu.VMEM((2 * H, D), jnp.float32),
            pltpu.VMEM((2 * H, D), jnp.bfloat16),
            pltpu.VMEM((H, D), jnp.float32),
            pltpu.VMEM((H, D), jnp.float32),
            pltpu.VMEM((E_loc, D, F), jnp.bfloat16),
            pltpu.VMEM((E_loc, F, D), jnp.bfloat16),
            pltpu.VMEM((2, Q1, F), jnp.float32),
            pltpu.VMEM((2, Q2, D), jnp.float32),
            pltpu.SemaphoreType.DMA((2,)),
            pltpu.SemaphoreType.DMA((2,)),
            pltpu.SemaphoreType.DMA((N_CHUNK,)),
            pltpu.SemaphoreType.DMA((N_CHUNK,)),
            pltpu.SemaphoreType.DMA((N_CHUNK,)),
            pltpu.SemaphoreType.DMA((N_CHUNK,)),
            pltpu.SemaphoreType.DMA((2,)),
            pltpu.SemaphoreType.DMA((2,)),
        ],
        compiler_params=pltpu.CompilerParams(
            collective_id=0, vmem_limit_bytes=63 * 1024 * 1024),
    )(x, router, W1, W2)


# device time: 65558 ns/iter; 1.1677x vs baseline; 1.1677x over previous
import jax
import jax.numpy as jnp
from jax import lax
from jax.experimental import pallas as pl
from jax.experimental.pallas import tpu as pltpu

NEG = -1e30
N_CHUNK = 4


def kernel(x, router, W1, W2):
    T, D = x.shape
    E_loc = router.shape[1]
    F = W1.shape[2]
    H = T // 2
    C = H // N_CHUNK

    def body(x_ref, r_ref, w1_hbm, w2_hbm, out_ref,
             r_recv, xcat, p_send, p_recv, w1_buf, w2_buf,
             sem_recv, sem_send, sem_pc, sem_ps, sem_oc, sem_os,
             sem_w1, sem_w2):
        my_x = lax.axis_index("x")
        my_y = lax.axis_index("y")
        xnbr = (1 - my_x, my_y)
        ynbr = (my_x, 1 - my_y)

        def start_load(e):
            slot = e % 2
            pltpu.make_async_copy(w1_hbm.at[e], w1_buf.at[slot],
                                  sem_w1.at[slot]).start()
            pltpu.make_async_copy(w2_hbm.at[e], w2_buf.at[slot],
                                  sem_w2.at[slot]).start()

        def wait_load(e):
            slot = e % 2
            pltpu.make_async_copy(w1_hbm.at[e], w1_buf.at[slot],
                                  sem_w1.at[slot]).wait()
            pltpu.make_async_copy(w2_hbm.at[e], w2_buf.at[slot],
                                  sem_w2.at[slot]).wait()

        start_load(0)
        start_load(1)

        xcat[pl.ds(0, H), :] = x_ref[pl.ds(my_y * H, H), :]

        barrier = pltpu.get_barrier_semaphore()
        for nbr in (xnbr, ynbr):
            pl.semaphore_signal(barrier, inc=1, device_id=nbr,
                                device_id_type=pl.DeviceIdType.MESH)
        pl.semaphore_wait(barrier, 2)

        rdma_r = pltpu.make_async_remote_copy(
            src_ref=r_ref, dst_ref=r_recv,
            send_sem=sem_send.at[0], recv_sem=sem_recv.at[0],
            device_id=xnbr, device_id_type=pl.DeviceIdType.MESH)
        rdma_r.start()
        rdma_x = pltpu.make_async_remote_copy(
            src_ref=xcat.at[pl.ds(0, H)], dst_ref=xcat.at[pl.ds(H, H)],
            send_sem=sem_send.at[1], recv_sem=sem_recv.at[1],
            device_id=xnbr, device_id_type=pl.DeviceIdType.MESH)
        rdma_x.start()
        rdma_r.wait()

        def gate_weights(xb):
            g_loc = jnp.dot(xb, r_ref[...], preferred_element_type=jnp.float32,
                            precision=lax.Precision.HIGHEST)
            g_rem = jnp.dot(xb, r_recv[...], preferred_element_type=jnp.float32,
                            precision=lax.Precision.HIGHEST)
            m1 = jnp.maximum(jnp.max(g_loc, 1, keepdims=True),
                             jnp.max(g_rem, 1, keepdims=True))
            is1 = g_loc == m1
            g2_loc = jnp.where(is1, NEG, g_loc)
            g2_rem = jnp.where(g_rem == m1, NEG, g_rem)
            m2 = jnp.maximum(jnp.max(g2_loc, 1, keepdims=True),
                             jnp.max(g2_rem, 1, keepdims=True))
            is2 = g2_loc == m2
            b = jnp.exp(m2 - m1)
            return (jnp.where(is1, 1.0 / (1.0 + b), 0.0)
                    + jnp.where(is2, b / (1.0 + b), 0.0))

        def ffn(xb, slot):
            h = jnp.maximum(
                jnp.dot(xb, w1_buf[slot], preferred_element_type=jnp.float32),
                0.0)
            return jnp.dot(h, w2_buf[slot], preferred_element_type=jnp.float32)

        w_lo = gate_weights(xcat[:H, :])

        wait_load(0)
        acc_lo = ffn(xcat[:H, :], 0) * w_lo[:, 0:1]
        rdma_x.wait()
        w_hi = gate_weights(xcat[H:, :])
        acc_hi = ffn(xcat[H:, :], 0) * w_hi[:, 0:1]
        start_load(2)

        wait_load(1)
        acc_lo = acc_lo + ffn(xcat[:H, :], 1) * w_lo[:, 1:2]
        acc_hi = acc_hi + ffn(xcat[H:, :], 1) * w_hi[:, 1:2]
        start_load(3)

        wait_load(2)
        acc_hi = acc_hi + ffn(xcat[H:, :], 2) * w_hi[:, 2:3]
        wait_load(3)
        acc_hi = acc_hi + ffn(xcat[H:, :], 3) * w_hi[:, 3:4]

        p_send[...] = acc_hi
        for c in range(N_CHUNK):
            pltpu.make_async_remote_copy(
                src_ref=p_send.at[pl.ds(c * C, C)],
                dst_ref=p_recv.at[pl.ds(c * C, C)],
                send_sem=sem_ps.at[c], recv_sem=sem_pc.at[c],
                device_id=xnbr, device_id_type=pl.DeviceIdType.MESH).start()

        acc_lo = acc_lo + ffn(xcat[:H, :], 2) * w_lo[:, 2:3]
        acc_lo = acc_lo + ffn(xcat[:H, :], 3) * w_lo[:, 3:4]

        fwd = []
        for c in range(N_CHUNK):
            rdma_p = pltpu.make_async_remote_copy(
                src_ref=p_send.at[pl.ds(c * C, C)],
                dst_ref=p_recv.at[pl.ds(c * C, C)],
                send_sem=sem_ps.at[c], recv_sem=sem_pc.at[c],
                device_id=xnbr, device_id_type=pl.DeviceIdType.MESH)
            rdma_p.wait_recv()
            rows = pl.ds(my_y * H + c * C, C)
            out_ref[rows, :] = (acc_lo[c * C:(c + 1) * C, :]
                                + p_recv[c * C:(c + 1) * C, :])
            rdma_o = pltpu.make_async_remote_copy(
                src_ref=out_ref.at[rows], dst_ref=out_ref.at[rows],
                send_sem=sem_os.at[c], recv_sem=sem_oc.at[c],
                device_id=ynbr, device_id_type=pl.DeviceIdType.MESH)
            rdma_o.start()
            fwd.append(rdma_o)

        for c in range(N_CHUNK):
            rdma_p = pltpu.make_async_remote_copy(
                src_ref=p_send.at[pl.ds(c * C, C)],
                dst_ref=p_recv.at[pl.ds(c * C, C)],
                send_sem=sem_ps.at[c], recv_sem=sem_pc.at[c],
                device_id=xnbr, device_id_type=pl.DeviceIdType.MESH)
            rdma_p.wait_send()
            fwd[c].wait()

    return pl.pallas_call(
        body,
        out_shape=jax.ShapeDtypeStruct((T, D), jnp.float32),
        in_specs=[
            pl.BlockSpec(memory_space=pltpu.VMEM),
            pl.BlockSpec(memory_space=pltpu.VMEM),
            pl.BlockSpec(memory_space=pl.ANY),
            pl.BlockSpec(memory_space=pl.ANY),
        ],
        out_specs=pl.BlockSpec(memory_space=pltpu.VMEM),
        scratch_shapes=[
            pltpu.VMEM((D, E_loc), jnp.float32),
            pltpu.VMEM((2 * H, D), jnp.float32),
            pltpu.VMEM((H, D), jnp.float32),
            pltpu.VMEM((H, D), jnp.float32),
            pltpu.VMEM((2, D, F), jnp.float32),
            pltpu.VMEM((2, F, D), jnp.float32),
            pltpu.SemaphoreType.DMA((2,)),
            pltpu.SemaphoreType.DMA((2,)),
            pltpu.SemaphoreType.DMA((N_CHUNK,)),
            pltpu.SemaphoreType.DMA((N_CHUNK,)),
            pltpu.SemaphoreType.DMA((N_CHUNK,)),
            pltpu.SemaphoreType.DMA((N_CHUNK,)),
            pltpu.SemaphoreType.DMA((2,)),
            pltpu.SemaphoreType.DMA((2,)),
        ],
        compiler_params=pltpu.CompilerParams(
            collective_id=0, vmem_limit_bytes=63 * 1024 * 1024),
    )(x, router, W1, W2)
